# baseline (speedup 1.0000x reference)
import functools

import jax
import jax.numpy as jnp
from jax import lax
from jax.experimental import pallas as pl
from jax.experimental.pallas import tpu as pltpu
from jax.experimental.pallas import tpu_sc as plsc

NC = 2
NS = 16
NW = NC * NS
LANES = 16
CHUNK = 128


def _sc_partials(x, src, dst):
    n, d = x.shape
    e = src.shape[0]
    n_chunks = e // CHUNK
    assert e % CHUNK == 0
    n_acc = ((n + NS * CHUNK - 1) // (NS * CHUNK)) * (NS * CHUNK)
    zchunks = n_acc // (NS * CHUNK)
    outer = (n_chunks + NW - 1) // NW
    outer4 = ((outer + 3) // 4) * 4

    ROWS_A = (n // NS) // 8 * 8
    ROWS_LAST = n - (NS - 1) * ROWS_A
    LAST_BASE = (NS - 1) * ROWS_A

    mesh = plsc.VectorSubcoreMesh(core_axis_name="c", subcore_axis_name="s")

    @functools.partial(
        pl.kernel,
        mesh=mesh,
        out_type=jax.ShapeDtypeStruct((NC, n, d), jnp.float32),
        scratch_types=[
            pltpu.VMEM_SHARED((n_acc, d), jnp.float32),
            pltpu.VMEM((CHUNK,), jnp.int32),
            pltpu.VMEM((CHUNK,), jnp.int32),
            pltpu.VMEM((CHUNK,), jnp.int32),
            pltpu.VMEM((CHUNK,), jnp.int32),
            pltpu.VMEM((CHUNK,), jnp.int32),
            pltpu.VMEM((CHUNK,), jnp.int32),
            pltpu.VMEM((CHUNK, d), jnp.float32),
            pltpu.VMEM((CHUNK, d), jnp.float32),
            pltpu.SemaphoreType.DMA,
            pltpu.SemaphoreType.DMA,
            pltpu.SemaphoreType.DMA,
            pltpu.SemaphoreType.DMA,
            pltpu.SemaphoreType.DMA,
            pltpu.SemaphoreType.DMA,
            pltpu.SemaphoreType.DMA,
            pltpu.SemaphoreType.DMA,
        ],
    )
    def sc_kernel(
        x_hbm, src_hbm, dst_hbm, out_hbm,
        acc, src_v0, src_v1, dst_v0, dst_v1, dst_v2, dst_v3, rows0, rows1,
        sem0, sem1, sem_s0, sem_s1, sem_d0, sem_d1, sem_d2, sem_d3,
    ):
        c = lax.axis_index("c")
        s = lax.axis_index("s")
        wid = s * NC + c
        gring = ((src_v0, rows0, sem0, sem_s0), (src_v1, rows1, sem1, sem_s1))
        dring = (
            (dst_v0, sem_d0), (dst_v1, sem_d1),
            (dst_v2, sem_d2), (dst_v3, sem_d3),
        )

        HA = ROWS_A // 2
        HL = ROWS_LAST // 2
        base = pl.multiple_of(s * ROWS_A, 8)

        @pl.when(c == 0)
        def _():
            @pl.when(s < NS - 1)
            def _():
                pltpu.async_copy(
                    x_hbm.at[pl.ds(base, HA)],
                    acc.at[pl.ds(base, HA)],
                    sem_s0,
                )
                pltpu.async_copy(
                    x_hbm.at[pl.ds(base + HA, HA)],
                    acc.at[pl.ds(base + HA, HA)],
                    sem_s1,
                )

            @pl.when(s == NS - 1)
            def _():
                pltpu.async_copy(
                    x_hbm.at[pl.ds(LAST_BASE, HL)],
                    acc.at[pl.ds(LAST_BASE, HL)],
                    sem_s0,
                )
                pltpu.async_copy(
                    x_hbm.at[pl.ds(LAST_BASE + HL, HL)],
                    acc.at[pl.ds(LAST_BASE + HL, HL)],
                    sem_s1,
                )

        @pl.when(c != 0)
        def _():
            @pl.loop(0, CHUNK)
            def _(i):
                @pl.loop(0, d // LANES)
                def _(j):
                    rows0[i, pl.ds(j * LANES, LANES)] = jnp.zeros(
                        (LANES,), jnp.float32
                    )

            @pl.loop(0, zchunks)
            def _(kk):
                pltpu.async_copy(
                    rows0,
                    acc.at[pl.ds((s * zchunks + kk) * CHUNK, CHUNK)],
                    sem_s0,
                )

        for t, (src_v, rows, sem, _sem_s) in enumerate(gring):
            pltpu.sync_copy(
                src_hbm.at[pl.ds((wid + t * NW) * CHUNK, CHUNK)], src_v
            )
        for v, (dst_v, sem_d) in enumerate(dring):
            pltpu.async_copy(
                dst_hbm.at[pl.ds((wid + v * NW) * CHUNK, CHUNK)], dst_v, sem_d
            )

        @pl.when(c == 0)
        def _():
            @pl.when(s < NS - 1)
            def _():
                pltpu.make_async_copy(
                    x_hbm.at[pl.ds(base, HA)], acc.at[pl.ds(base, HA)], sem_s0
                ).wait()
                pltpu.make_async_copy(
                    x_hbm.at[pl.ds(base + HA, HA)],
                    acc.at[pl.ds(base + HA, HA)],
                    sem_s1,
                ).wait()

            @pl.when(s == NS - 1)
            def _():
                pltpu.make_async_copy(
                    x_hbm.at[pl.ds(LAST_BASE, HL)],
                    acc.at[pl.ds(LAST_BASE, HL)],
                    sem_s0,
                ).wait()
                pltpu.make_async_copy(
                    x_hbm.at[pl.ds(LAST_BASE + HL, HL)],
                    acc.at[pl.ds(LAST_BASE + HL, HL)],
                    sem_s1,
                ).wait()

        @pl.when(c != 0)
        def _():
            @pl.loop(0, zchunks)
            def _(kk):
                pltpu.make_async_copy(
                    rows0,
                    acc.at[pl.ds((s * zchunks + kk) * CHUNK, CHUNK)],
                    sem_s0,
                ).wait()

        for t, (src_v, rows, sem, _sem_s) in enumerate(gring):
            pltpu.async_copy(x_hbm.at[src_v], rows, sem)

        plsc.subcore_barrier()

        @pl.loop(0, outer4, step=4)
        def _(j):
            for v in range(4):
                src_v, rows, sem, sem_s = gring[v % 2]
                dst_v, sem_d = dring[v]
                chunk = wid + (j + v) * NW

                @pl.when(chunk < n_chunks)
                def _():
                    nxt = chunk + 2 * NW
                    nxt4 = chunk + 4 * NW

                    pltpu.make_async_copy(x_hbm.at[src_v], rows, sem).wait()

                    @pl.when(nxt < n_chunks)
                    def _():
                        pltpu.async_copy(
                            src_hbm.at[pl.ds(nxt * CHUNK, CHUNK)],
                            src_v,
                            sem_s,
                        )

                    pltpu.make_async_copy(
                        dst_hbm.at[pl.ds(chunk * CHUNK, CHUNK)], dst_v, sem_d
                    ).wait()
                    pltpu.sync_copy(rows, acc.at[dst_v], add=True)

                    @pl.when(nxt < n_chunks)
                    def _():
                        pltpu.make_async_copy(
                            src_hbm.at[pl.ds(nxt * CHUNK, CHUNK)],
                            src_v,
                            sem_s,
                        ).wait()
                        pltpu.async_copy(x_hbm.at[src_v], rows, sem)

                    @pl.when(nxt4 < n_chunks)
                    def _():
                        pltpu.async_copy(
                            dst_hbm.at[pl.ds(nxt4 * CHUNK, CHUNK)],
                            dst_v,
                            sem_d,
                        )

        plsc.subcore_barrier()

        wbase = pl.multiple_of(s * ROWS_A, 8)

        @pl.when(s < NS - 1)
        def _():
            pltpu.sync_copy(
                acc.at[pl.ds(wbase, ROWS_A)],
                out_hbm.at[c].at[pl.ds(wbase, ROWS_A)],
            )

        @pl.when(s == NS - 1)
        def _():
            pltpu.sync_copy(
                acc.at[pl.ds(LAST_BASE, ROWS_LAST)],
                out_hbm.at[c].at[pl.ds(LAST_BASE, ROWS_LAST)],
            )

    return sc_kernel(x, src, dst)


def _tc_linear(p, wt, b):
    _, n, d = p.shape
    tm = 1000
    assert n % tm == 0

    def mm_kernel(p_ref, wt_ref, b_ref, o_ref):
        h = p_ref[0] + p_ref[1]
        o_ref[...] = (
            jnp.dot(h, wt_ref[...], preferred_element_type=jnp.float32)
            + b_ref[...]
        )

    return pl.pallas_call(
        mm_kernel,
        grid=(n // tm,),
        in_specs=[
            pl.BlockSpec((NC, tm, d), lambda i: (0, i, 0)),
            pl.BlockSpec((d, d), lambda i: (0, 0)),
            pl.BlockSpec((1, d), lambda i: (0, 0)),
        ],
        out_specs=pl.BlockSpec((tm, d), lambda i: (i, 0)),
        out_shape=jax.ShapeDtypeStruct((n, d), jnp.float32),
    )(p, wt, b.reshape(1, d))


def kernel(x, edge_index, edge_weight, W, b):
    src = edge_index[0].astype(jnp.int32)
    dst = edge_index[1].astype(jnp.int32)
    partials = _sc_partials(x, src, dst)
    return _tc_linear(partials, W.T, b)


# speedup vs baseline: 1.0169x; 1.0169x over previous
import functools

import jax
import jax.numpy as jnp
from jax import lax
from jax.experimental import pallas as pl
from jax.experimental.pallas import tpu as pltpu
from jax.experimental.pallas import tpu_sc as plsc

NC = 2
NS = 16
NW = NC * NS
LANES = 16
CHUNK = 128


def _sc_partials(x, src, dst):
    n, d = x.shape
    e = src.shape[0]
    n_chunks = e // CHUNK
    assert e % CHUNK == 0
    n_acc = ((n + NS * CHUNK - 1) // (NS * CHUNK)) * (NS * CHUNK)
    zchunks = n_acc // (NS * CHUNK)
    outer = (n_chunks + NW - 1) // NW
    outer4 = ((outer + 3) // 4) * 4

    ROWS_A = (n // NS) // 8 * 8
    ROWS_LAST = n - (NS - 1) * ROWS_A
    LAST_BASE = (NS - 1) * ROWS_A

    mesh = plsc.VectorSubcoreMesh(core_axis_name="c", subcore_axis_name="s")

    @functools.partial(
        pl.kernel,
        mesh=mesh,
        out_type=jax.ShapeDtypeStruct((NC, n, d), jnp.float32),
        scratch_types=[
            pltpu.VMEM_SHARED((n_acc, d), jnp.float32),
            pltpu.VMEM((CHUNK,), jnp.int32),
            pltpu.VMEM((CHUNK,), jnp.int32),
            pltpu.VMEM((CHUNK,), jnp.int32),
            pltpu.VMEM((CHUNK,), jnp.int32),
            pltpu.VMEM((CHUNK,), jnp.int32),
            pltpu.VMEM((CHUNK,), jnp.int32),
            pltpu.VMEM((CHUNK, d), jnp.float32),
            pltpu.VMEM((CHUNK, d), jnp.float32),
            pltpu.SemaphoreType.DMA,
            pltpu.SemaphoreType.DMA,
            pltpu.SemaphoreType.DMA,
            pltpu.SemaphoreType.DMA,
            pltpu.SemaphoreType.DMA,
            pltpu.SemaphoreType.DMA,
            pltpu.SemaphoreType.DMA,
            pltpu.SemaphoreType.DMA,
        ],
    )
    def sc_kernel(
        x_hbm, src_hbm, dst_hbm, out_hbm,
        acc, src_v0, src_v1, dst_v0, dst_v1, dst_v2, dst_v3, rows0, rows1,
        sem0, sem1, sem_s0, sem_s1, sem_d0, sem_d1, sem_d2, sem_d3,
    ):
        c = lax.axis_index("c")
        s = lax.axis_index("s")
        wid = s * NC + c
        gring = ((src_v0, rows0, sem0, sem_s0), (src_v1, rows1, sem1, sem_s1))
        dring = (
            (dst_v0, sem_d0), (dst_v1, sem_d1),
            (dst_v2, sem_d2), (dst_v3, sem_d3),
        )

        HA = ROWS_A // 2
        HL = ROWS_LAST // 2
        base = pl.multiple_of(s * ROWS_A, 8)

        @pl.when(c == 0)
        def _():
            @pl.when(s < NS - 1)
            def _():
                pltpu.async_copy(
                    x_hbm.at[pl.ds(base, HA)],
                    acc.at[pl.ds(base, HA)],
                    sem_s0,
                )
                pltpu.async_copy(
                    x_hbm.at[pl.ds(base + HA, HA)],
                    acc.at[pl.ds(base + HA, HA)],
                    sem_s1,
                )

            @pl.when(s == NS - 1)
            def _():
                pltpu.async_copy(
                    x_hbm.at[pl.ds(LAST_BASE, HL)],
                    acc.at[pl.ds(LAST_BASE, HL)],
                    sem_s0,
                )
                pltpu.async_copy(
                    x_hbm.at[pl.ds(LAST_BASE + HL, HL)],
                    acc.at[pl.ds(LAST_BASE + HL, HL)],
                    sem_s1,
                )

        @pl.when(c != 0)
        def _():
            @pl.loop(0, CHUNK)
            def _(i):
                @pl.loop(0, d // LANES)
                def _(j):
                    rows0[i, pl.ds(j * LANES, LANES)] = jnp.zeros(
                        (LANES,), jnp.float32
                    )

            @pl.loop(0, zchunks)
            def _(kk):
                pltpu.async_copy(
                    rows0,
                    acc.at[pl.ds((s * zchunks + kk) * CHUNK, CHUNK)],
                    sem_s0,
                )

        for t, (src_v, rows, sem, _sem_s) in enumerate(gring):
            pltpu.sync_copy(
                src_hbm.at[pl.ds((wid + t * NW) * CHUNK, CHUNK)], src_v
            )
        for v, (dst_v, sem_d) in enumerate(dring):
            pltpu.async_copy(
                dst_hbm.at[pl.ds((wid + v * NW) * CHUNK, CHUNK)], dst_v, sem_d
            )

        @pl.when(c == 0)
        def _():
            @pl.when(s < NS - 1)
            def _():
                pltpu.make_async_copy(
                    x_hbm.at[pl.ds(base, HA)], acc.at[pl.ds(base, HA)], sem_s0
                ).wait()
                pltpu.make_async_copy(
                    x_hbm.at[pl.ds(base + HA, HA)],
                    acc.at[pl.ds(base + HA, HA)],
                    sem_s1,
                ).wait()

            @pl.when(s == NS - 1)
            def _():
                pltpu.make_async_copy(
                    x_hbm.at[pl.ds(LAST_BASE, HL)],
                    acc.at[pl.ds(LAST_BASE, HL)],
                    sem_s0,
                ).wait()
                pltpu.make_async_copy(
                    x_hbm.at[pl.ds(LAST_BASE + HL, HL)],
                    acc.at[pl.ds(LAST_BASE + HL, HL)],
                    sem_s1,
                ).wait()

        @pl.when(c != 0)
        def _():
            @pl.loop(0, zchunks)
            def _(kk):
                pltpu.make_async_copy(
                    rows0,
                    acc.at[pl.ds((s * zchunks + kk) * CHUNK, CHUNK)],
                    sem_s0,
                ).wait()

        for t, (src_v, rows, sem, _sem_s) in enumerate(gring):
            pltpu.async_copy(x_hbm.at[src_v], rows, sem)

        plsc.subcore_barrier()

        @pl.loop(0, outer4, step=4)
        def _(j):
            for v in range(4):
                src_v, rows, sem, sem_s = gring[v % 2]
                dst_v, sem_d = dring[v]
                chunk = wid + (j + v) * NW

                @pl.when(chunk < n_chunks)
                def _():
                    nxt = chunk + 2 * NW
                    nxt4 = chunk + 4 * NW

                    pltpu.make_async_copy(x_hbm.at[src_v], rows, sem).wait()

                    @pl.when(nxt < n_chunks)
                    def _():
                        pltpu.async_copy(
                            src_hbm.at[pl.ds(nxt * CHUNK, CHUNK)],
                            src_v,
                            sem_s,
                        )

                    pltpu.make_async_copy(
                        dst_hbm.at[pl.ds(chunk * CHUNK, CHUNK)], dst_v, sem_d
                    ).wait()
                    pltpu.sync_copy(rows, acc.at[dst_v], add=True)

                    @pl.when(nxt < n_chunks)
                    def _():
                        pltpu.make_async_copy(
                            src_hbm.at[pl.ds(nxt * CHUNK, CHUNK)],
                            src_v,
                            sem_s,
                        ).wait()
                        pltpu.async_copy(x_hbm.at[src_v], rows, sem)

                    @pl.when(nxt4 < n_chunks)
                    def _():
                        pltpu.async_copy(
                            dst_hbm.at[pl.ds(nxt4 * CHUNK, CHUNK)],
                            dst_v,
                            sem_d,
                        )

        plsc.subcore_barrier()

        wbase = pl.multiple_of(s * ROWS_A, 8)

        @pl.when(s < NS - 1)
        def _():
            pltpu.sync_copy(
                acc.at[pl.ds(wbase, ROWS_A)],
                out_hbm.at[c].at[pl.ds(wbase, ROWS_A)],
            )

        @pl.when(s == NS - 1)
        def _():
            pltpu.sync_copy(
                acc.at[pl.ds(LAST_BASE, ROWS_LAST)],
                out_hbm.at[c].at[pl.ds(LAST_BASE, ROWS_LAST)],
            )

    return sc_kernel(x, src, dst)


def _tc_linear(p, wt, b):
    _, n, d = p.shape
    tm = 2000
    assert n % tm == 0

    def mm_kernel(p_ref, wt_ref, b_ref, o_ref):
        h = p_ref[0] + p_ref[1]
        o_ref[...] = (
            jnp.dot(h, wt_ref[...], preferred_element_type=jnp.float32)
            + b_ref[...]
        )

    return pl.pallas_call(
        mm_kernel,
        grid=(n // tm,),
        in_specs=[
            pl.BlockSpec((NC, tm, d), lambda i: (0, i, 0)),
            pl.BlockSpec((d, d), lambda i: (0, 0)),
            pl.BlockSpec((1, d), lambda i: (0, 0)),
        ],
        out_specs=pl.BlockSpec((tm, d), lambda i: (i, 0)),
        out_shape=jax.ShapeDtypeStruct((n, d), jnp.float32),
    )(p, wt, b.reshape(1, d))


def kernel(x, edge_index, edge_weight, W, b):
    src = edge_index[0].astype(jnp.int32)
    dst = edge_index[1].astype(jnp.int32)
    partials = _sc_partials(x, src, dst)
    return _tc_linear(partials, W.T, b)
